# Initial kernel scaffold; baseline (speedup 1.0000x reference)
import jax
import jax.numpy as jnp
from jax.experimental import pallas as pl


def kernel(x, W, b, A, Bm, LiMEs):
    raise NotImplementedError("write your pallas kernel here")


# speedup vs baseline: 5.5144x; 5.5144x over previous
import functools

import jax
import jax.numpy as jnp
from jax.experimental import pallas as pl
from jax.experimental.pallas import tpu as pltpu

E = 64
K = 8
R = 16
TEMP = 0.5
EPS = 1e-6
TILE = 512


def _phase1_kernel(x_ref, ws_ref, h_ref, max_ref):
    h = jax.lax.dot_general(
        x_ref[:], ws_ref[:],
        dimension_numbers=(((1,), (1,)), ((), ())),
        preferred_element_type=jnp.float32,
    )
    h_ref[:] = h
    tile_max = jnp.max(jnp.abs(h))

    @pl.when(pl.program_id(0) == 0)
    def _init():
        max_ref[0, 0] = tile_max

    @pl.when(pl.program_id(0) != 0)
    def _acc():
        max_ref[0, 0] = jnp.maximum(max_ref[0, 0], tile_max)


def _phase2_kernel(scale_ref, x_ref, h_ref, w_ref, b_ref, a_ref, bm_ref,
                   limes_ref, out_ref):
    x = x_ref[:]
    base = jax.lax.dot_general(
        x, w_ref[:],
        dimension_numbers=(((1,), (1,)), ((), ())),
        preferred_element_type=jnp.float32,
    ) + b_ref[:]

    scale = jnp.maximum(scale_ref[0, 0], EPS)
    inv = 1.0 / (scale * TEMP)
    logits = h_ref[:] * inv
    m = jnp.max(logits, axis=-1, keepdims=True)
    e = jnp.exp(logits - m)
    ii = jax.lax.broadcasted_iota(jnp.int32, e.shape, 1)

    masked = e
    wmat = jnp.zeros_like(e)
    ssum = jnp.zeros((e.shape[0], 1), jnp.float32)
    for _ in range(K):
        cur = jnp.max(masked, axis=-1, keepdims=True)
        ismax = masked == cur
        midx = jnp.where(ismax, ii, E)
        first = midx == jnp.min(midx, axis=-1, keepdims=True)
        wmat = wmat + jnp.where(first, masked, 0.0)
        ssum = ssum + cur
        masked = jnp.where(first, -1.0, masked)

    w = wmat / ssum
    p_mix = jnp.dot(w, limes_ref[:], preferred_element_type=jnp.float32)

    u = jnp.dot(x, a_ref[:], preferred_element_type=jnp.float32)
    delta = jnp.dot(u * p_mix, bm_ref[:], preferred_element_type=jnp.float32)
    out_ref[:] = base + delta


def kernel(x, W, b, A, Bm, LiMEs):
    B, T, d_in = x.shape
    d_out = W.shape[0]
    n_tok = B * T
    nt = n_tok // TILE
    x2 = x.reshape(n_tok, d_in)

    h, mx = pl.pallas_call(
        _phase1_kernel,
        grid=(nt,),
        in_specs=[
            pl.BlockSpec((TILE, d_in), lambda i: (i, 0)),
            pl.BlockSpec((E, d_in), lambda i: (0, 0)),
        ],
        out_specs=[
            pl.BlockSpec((TILE, E), lambda i: (i, 0)),
            pl.BlockSpec((1, 1), lambda i: (0, 0), memory_space=pltpu.SMEM),
        ],
        out_shape=[
            jax.ShapeDtypeStruct((n_tok, E), jnp.float32),
            jax.ShapeDtypeStruct((1, 1), jnp.float32),
        ],
        compiler_params=pltpu.CompilerParams(
            dimension_semantics=("arbitrary",)),
    )(x2, W)

    out = pl.pallas_call(
        _phase2_kernel,
        grid=(nt,),
        in_specs=[
            pl.BlockSpec(memory_space=pltpu.SMEM),
            pl.BlockSpec((TILE, d_in), lambda i: (i, 0)),
            pl.BlockSpec((TILE, E), lambda i: (i, 0)),
            pl.BlockSpec((d_out, d_in), lambda i: (0, 0)),
            pl.BlockSpec((1, d_out), lambda i: (0, 0)),
            pl.BlockSpec((d_in, R), lambda i: (0, 0)),
            pl.BlockSpec((R, d_out), lambda i: (0, 0)),
            pl.BlockSpec((E, R), lambda i: (0, 0)),
        ],
        out_specs=pl.BlockSpec((TILE, d_out), lambda i: (i, 0)),
        out_shape=jax.ShapeDtypeStruct((n_tok, d_out), jnp.float32),
        compiler_params=pltpu.CompilerParams(
            dimension_semantics=("arbitrary",)),
    )(mx, x2, h, W, b.reshape(1, d_out), A, Bm, LiMEs)

    return out.reshape(B, T, d_out)
